# baseline (speedup 1.0000x reference)
import functools

import numpy as np

import jax
import jax.numpy as jnp
from jax import lax
from jax.experimental import pallas as pl
from jax.experimental.pallas import tpu as pltpu

_NUM_DRAFTS = 4
_DRAFT_LENGTH = 8
_NUM_SAMPLES = _NUM_DRAFTS * _DRAFT_LENGTH
_GROUPS = 2

_SUBKEYS = np.array(
    [
        [64467757, 2916123636], [1705926158, 899080142],
        [1712723395, 2526649282], [2232176465, 33846082],
        [767915537, 735759787], [2252301940, 331845914],
        [2395792924, 649865367], [3515226245, 1150219387],
        [1308905690, 3242231867], [3647288517, 4265293960],
        [3122727659, 270479714], [2427900899, 967170697],
        [1632469386, 2634931831], [2599281759, 1608025340],
        [2169252222, 555766829], [1364423604, 2995594396],
        [36762624, 630778214], [465666600, 414652409],
        [3487786411, 1377120945], [2152688743, 2163763172],
        [687918034, 1158374039], [2077459142, 3925410892],
        [3206933193, 1369254882], [1603636283, 2173888969],
        [763496773, 728625621], [728859045, 2443894681],
        [2914324375, 3880442531], [2305633233, 858993948],
        [2761967522, 475394688], [1647933109, 3161123727],
        [2125958545, 2473280340], [3352461584, 1728412684],
    ],
    dtype=np.uint32,
)

_ROT_A = (13, 15, 26, 6)
_ROT_B = (17, 29, 16, 24)
_TINY = np.float32(np.finfo(np.float32).tiny)
_NEG_INF = np.float32(-np.inf)


def _rotl(x, r):
    r = np.uint32(r)
    return (x << r) | (x >> np.uint32(32 - r))


def _threefry_bits(k0, k1, x1):
    ks2 = k0 ^ k1 ^ np.uint32(0x1BD11BDA)
    ks = (k0, k1, ks2)
    x0 = jnp.full_like(x1, 0) + k0
    x1 = x1 + k1
    for i in range(5):
        rots = _ROT_A if i % 2 == 0 else _ROT_B
        for r in rots:
            x0 = x0 + x1
            x1 = _rotl(x1, r)
            x1 = x1 ^ x0
        x0 = x0 + ks[(i + 1) % 3]
        x1 = x1 + ks[(i + 2) % 3] + np.uint32(i + 1)
    return x0 ^ x1


def _gumbel_from_bits(bits):
    fb = (bits >> np.uint32(9)) | np.uint32(0x3F800000)
    f = lax.bitcast_convert_type(fb, jnp.float32) - np.float32(1.0)
    u = jnp.maximum(_TINY, f * (np.float32(1.0) - _TINY) + _TINY)
    return -jnp.log(-jnp.log(u))


def _sampler_kernel(keys_ref, logits_ref, tok_ref, prob_ref,
                    sums_ref, rmax_ref, rarg_ref,
                    *, vocab, chunk, nchunks, batch, spg):
    g = pl.program_id(0)
    c = pl.program_id(1)

    @pl.when(c == 0)
    def _init():
        sums_ref[...] = jnp.zeros_like(sums_ref)
        rmax_ref[...] = jnp.full_like(rmax_ref, _NEG_INF)
        rarg_ref[...] = jnp.zeros_like(rarg_ref)

    l = logits_ref[...]
    col = lax.broadcasted_iota(jnp.int32, (batch, chunk), 1) + c * chunk
    mask = col < vocab
    lm = jnp.where(mask, l, _NEG_INF)

    sums_ref[...] += jnp.exp(lm)

    row = lax.broadcasted_iota(jnp.int32, (batch, chunk), 0)
    idx = (row * vocab + col).astype(jnp.uint32)

    lane = lax.broadcasted_iota(jnp.int32, (batch, spg), 1)

    def one_sample(s):
        k0 = keys_ref[g * spg + s, 0].astype(jnp.uint32)
        k1 = keys_ref[g * spg + s, 1].astype(jnp.uint32)
        bits = _threefry_bits(k0, k1, idx)
        val = lm + _gumbel_from_bits(bits)
        rowmax = jnp.max(val, axis=1, keepdims=True)
        eq = val == rowmax
        rowarg = jnp.min(jnp.where(eq, col, jnp.int32(0x7FFFFFFF)),
                         axis=1, keepdims=True)
        return rowmax, rowarg

    def sample_body(j, carry):
        cmax, carg = carry
        s0 = 8 * j
        for t in range(8):
            m, a = one_sample(s0 + t)
            sel = lane == s0 + t
            cmax = jnp.where(sel, m, cmax)
            carg = jnp.where(sel, a, carg)
        return cmax, carg

    init = (jnp.full((batch, spg), _NEG_INF, jnp.float32),
            jnp.zeros((batch, spg), jnp.int32))
    cmax, carg = lax.fori_loop(0, spg // 8, sample_body, init)

    gt = cmax > rmax_ref[...]
    rmax_ref[...] = jnp.where(gt, cmax, rmax_ref[...])
    rarg_ref[...] = jnp.where(gt, carg, rarg_ref[...])

    @pl.when(c == nchunks - 1)
    def _finish():
        total = jnp.sum(sums_ref[...], axis=1, keepdims=True)
        tok = rarg_ref[...]
        brow = lax.broadcasted_iota(jnp.int32, (batch, spg), 0)
        tidx = (brow * vocab + tok).astype(jnp.uint32)

        def key_body(s, acc):
            k0 = keys_ref[g * spg + s, 0].astype(jnp.uint32)
            k1 = keys_ref[g * spg + s, 1].astype(jnp.uint32)
            gtok = _gumbel_from_bits(_threefry_bits(k0, k1, tidx))
            return jnp.where(lane == s, gtok, acc)

        gsel = lax.fori_loop(0, spg, key_body, jnp.zeros((batch, spg), jnp.float32))
        tok_ref[0] = tok
        prob_ref[0] = jnp.exp(rmax_ref[...] - gsel) / total


def _run_sampler(last_logits):
    batch, vocab = last_logits.shape
    chunk = 3840 if vocab >= 3840 else ((vocab + 127) // 128) * 128
    nchunks = -(-vocab // chunk)
    spg = _NUM_SAMPLES // _GROUPS

    body = functools.partial(_sampler_kernel, vocab=vocab, chunk=chunk,
                             nchunks=nchunks, batch=batch, spg=spg)
    keys = jnp.asarray(_SUBKEYS.view(np.int32))
    tok, prob = pl.pallas_call(
        body,
        grid=(_GROUPS, nchunks),
        in_specs=[
            pl.BlockSpec(memory_space=pltpu.SMEM),
            pl.BlockSpec((batch, chunk), lambda g, i: (0, i)),
        ],
        out_specs=[
            pl.BlockSpec((1, batch, spg), lambda g, i: (g, 0, 0)),
            pl.BlockSpec((1, batch, spg), lambda g, i: (g, 0, 0)),
        ],
        out_shape=[
            jax.ShapeDtypeStruct((_GROUPS, batch, spg), jnp.int32),
            jax.ShapeDtypeStruct((_GROUPS, batch, spg), jnp.float32),
        ],
        scratch_shapes=[
            pltpu.VMEM((batch, chunk), jnp.float32),
            pltpu.VMEM((batch, spg), jnp.float32),
            pltpu.VMEM((batch, spg), jnp.int32),
        ],
        compiler_params=pltpu.CompilerParams(
            dimension_semantics=("parallel", "arbitrary"),
        ),
    )(keys, last_logits)
    tok = tok.transpose(1, 0, 2).reshape(batch, _NUM_SAMPLES)
    prob = prob.transpose(1, 0, 2).reshape(batch, _NUM_SAMPLES)
    return tok, prob


def kernel(hidden_states, logits, W, b):
    del hidden_states, W, b
    batch = logits.shape[0]
    last_logits = logits[:, -1, :]
    tok, prob = _run_sampler(last_logits)
    return (tok.reshape(batch, _NUM_DRAFTS, _DRAFT_LENGTH),
            prob.reshape(batch, _NUM_DRAFTS, _DRAFT_LENGTH))


# speedup vs baseline: 1.1107x; 1.1107x over previous
import functools

import numpy as np

import jax
import jax.numpy as jnp
from jax import lax
from jax.experimental import pallas as pl
from jax.experimental.pallas import tpu as pltpu

_NUM_DRAFTS = 4
_DRAFT_LENGTH = 8
_NUM_SAMPLES = _NUM_DRAFTS * _DRAFT_LENGTH
_GROUPS = 2

_SUBKEYS = np.array(
    [
        [64467757, 2916123636], [1705926158, 899080142],
        [1712723395, 2526649282], [2232176465, 33846082],
        [767915537, 735759787], [2252301940, 331845914],
        [2395792924, 649865367], [3515226245, 1150219387],
        [1308905690, 3242231867], [3647288517, 4265293960],
        [3122727659, 270479714], [2427900899, 967170697],
        [1632469386, 2634931831], [2599281759, 1608025340],
        [2169252222, 555766829], [1364423604, 2995594396],
        [36762624, 630778214], [465666600, 414652409],
        [3487786411, 1377120945], [2152688743, 2163763172],
        [687918034, 1158374039], [2077459142, 3925410892],
        [3206933193, 1369254882], [1603636283, 2173888969],
        [763496773, 728625621], [728859045, 2443894681],
        [2914324375, 3880442531], [2305633233, 858993948],
        [2761967522, 475394688], [1647933109, 3161123727],
        [2125958545, 2473280340], [3352461584, 1728412684],
    ],
    dtype=np.uint32,
)

_ROT_A = (13, 15, 26, 6)
_ROT_B = (17, 29, 16, 24)
_TINY = np.float32(np.finfo(np.float32).tiny)
_NEG_INF = np.float32(-np.inf)


def _rotl(x, r):
    r = np.uint32(r)
    return (x << r) | (x >> np.uint32(32 - r))


def _threefry_bits(k0, k1, x1):
    ks2 = k0 ^ k1 ^ np.uint32(0x1BD11BDA)
    ks = (k0, k1, ks2)
    x0 = jnp.full_like(x1, 0) + k0
    x1 = x1 + k1
    for i in range(5):
        rots = _ROT_A if i % 2 == 0 else _ROT_B
        for r in rots:
            x0 = x0 + x1
            x1 = _rotl(x1, r)
            x1 = x1 ^ x0
        x0 = x0 + ks[(i + 1) % 3]
        x1 = x1 + ks[(i + 2) % 3] + np.uint32(i + 1)
    return x0 ^ x1


def _gumbel_from_bits(bits):
    fb = (bits >> np.uint32(9)) | np.uint32(0x3F800000)
    f = lax.bitcast_convert_type(fb, jnp.float32) - np.float32(1.0)
    u = jnp.maximum(_TINY, f * (np.float32(1.0) - _TINY) + _TINY)
    return -jnp.log(-jnp.log(u))


def _sampler_kernel(keys_ref, logits_ref, tok_ref, prob_ref,
                    sums_ref, rmax_ref, rarg_ref,
                    *, vocab, chunk, nchunks, batch, spg):
    g = pl.program_id(0)
    c = pl.program_id(1)

    @pl.when(c == 0)
    def _init():
        sums_ref[...] = jnp.zeros_like(sums_ref)
        rmax_ref[...] = jnp.full_like(rmax_ref, _NEG_INF)
        rarg_ref[...] = jnp.zeros_like(rarg_ref)

    l = logits_ref[...]
    col = lax.broadcasted_iota(jnp.int32, (batch, chunk), 1) + c * chunk
    mask = col < vocab
    lm = jnp.where(mask, l, _NEG_INF)

    sums_ref[...] += jnp.exp(lm)

    row = lax.broadcasted_iota(jnp.int32, (batch, chunk), 0)
    idx = (row * vocab + col).astype(jnp.uint32)

    lane = lax.broadcasted_iota(jnp.int32, (batch, spg), 1)

    def one_sample(s):
        k0 = keys_ref[g * spg + s, 0].astype(jnp.uint32)
        k1 = keys_ref[g * spg + s, 1].astype(jnp.uint32)
        bits = _threefry_bits(k0, k1, idx)
        val = lm + _gumbel_from_bits(bits)
        rowmax = jnp.max(val, axis=1, keepdims=True)
        eq = val == rowmax
        rowarg = jnp.min(jnp.where(eq, col, jnp.int32(0x7FFFFFFF)),
                         axis=1, keepdims=True)
        return rowmax, rowarg

    def sample_body(j, carry):
        cmax, carg = carry
        s0 = 8 * j
        for t in range(8):
            m, a = one_sample(s0 + t)
            sel = lane == s0 + t
            cmax = jnp.where(sel, m, cmax)
            carg = jnp.where(sel, a, carg)
        return cmax, carg

    init = (jnp.full((batch, spg), _NEG_INF, jnp.float32),
            jnp.zeros((batch, spg), jnp.int32))
    cmax, carg = lax.fori_loop(0, spg // 8, sample_body, init)

    gt = cmax > rmax_ref[...]
    rmax_ref[...] = jnp.where(gt, cmax, rmax_ref[...])
    rarg_ref[...] = jnp.where(gt, carg, rarg_ref[...])

    @pl.when(c == nchunks - 1)
    def _finish():
        total = jnp.sum(sums_ref[...], axis=1, keepdims=True)
        tok = rarg_ref[...]
        brow = lax.broadcasted_iota(jnp.int32, (batch, spg), 0)
        tidx = (brow * vocab + tok).astype(jnp.uint32)

        def key_body(s, acc):
            k0 = keys_ref[g * spg + s, 0].astype(jnp.uint32)
            k1 = keys_ref[g * spg + s, 1].astype(jnp.uint32)
            gtok = _gumbel_from_bits(_threefry_bits(k0, k1, tidx))
            return jnp.where(lane == s, gtok, acc)

        gsel = lax.fori_loop(0, spg, key_body, jnp.zeros((batch, spg), jnp.float32))
        tok_ref[0] = tok
        prob_ref[0] = jnp.exp(rmax_ref[...] - gsel) / total


def _run_sampler(last_logits):
    batch, vocab = last_logits.shape
    chunk = 3584 if vocab >= 3584 else ((vocab + 127) // 128) * 128
    nchunks = -(-vocab // chunk)
    spg = _NUM_SAMPLES // _GROUPS

    body = functools.partial(_sampler_kernel, vocab=vocab, chunk=chunk,
                             nchunks=nchunks, batch=batch, spg=spg)
    keys = jnp.asarray(_SUBKEYS.view(np.int32))
    tok, prob = pl.pallas_call(
        body,
        grid=(_GROUPS, nchunks),
        in_specs=[
            pl.BlockSpec(memory_space=pltpu.SMEM),
            pl.BlockSpec((batch, chunk), lambda g, i: (0, i)),
        ],
        out_specs=[
            pl.BlockSpec((1, batch, spg), lambda g, i: (g, 0, 0)),
            pl.BlockSpec((1, batch, spg), lambda g, i: (g, 0, 0)),
        ],
        out_shape=[
            jax.ShapeDtypeStruct((_GROUPS, batch, spg), jnp.int32),
            jax.ShapeDtypeStruct((_GROUPS, batch, spg), jnp.float32),
        ],
        scratch_shapes=[
            pltpu.VMEM((batch, chunk), jnp.float32),
            pltpu.VMEM((batch, spg), jnp.float32),
            pltpu.VMEM((batch, spg), jnp.int32),
        ],
        compiler_params=pltpu.CompilerParams(
            dimension_semantics=("parallel", "arbitrary"),
        ),
    )(keys, last_logits)
    tok = tok.transpose(1, 0, 2).reshape(batch, _NUM_SAMPLES)
    prob = prob.transpose(1, 0, 2).reshape(batch, _NUM_SAMPLES)
    return tok, prob


def kernel(hidden_states, logits, W, b):
    del hidden_states, W, b
    batch = logits.shape[0]
    last_logits = logits[:, -1, :]
    tok, prob = _run_sampler(last_logits)
    return (tok.reshape(batch, _NUM_DRAFTS, _DRAFT_LENGTH),
            prob.reshape(batch, _NUM_DRAFTS, _DRAFT_LENGTH))
